# Initial kernel scaffold; baseline (speedup 1.0000x reference)
import jax
import jax.numpy as jnp
from jax.experimental import pallas as pl


def kernel(x, edge_index, batch, W1, b1, W2, b2, Wfc, bfc):
    raise NotImplementedError("write your pallas kernel here")


# speedup vs baseline: 7.9254x; 7.9254x over previous
import functools

import jax
import jax.numpy as jnp
from jax import lax
from jax.experimental import pallas as pl
from jax.experimental.pallas import tpu as pltpu
from jax.experimental.pallas import tpu_sc as plsc

N = 10000
E = 320000
D = 128
G = 16

NC = 2
NS = 16
NW = NC * NS

K = 128
CHUNKS = 80
EP = NW * CHUNKS * K
NPAD = 10240
ZR = NPAD // NS
ZC = ZR // K

BLK = 1000


def _deg_body(dst_hbm, zid_hbm, zeros_hbm, ones_hbm, out_hbm,
              dst_v, zid_v, zrows_v, ones_v, acc_sh):
    c = lax.axis_index("c")
    s = lax.axis_index("s")
    wid = c * NS + s
    pltpu.sync_copy(dst_hbm.at[wid], dst_v)
    pltpu.sync_copy(zid_hbm.at[s], zid_v)
    pltpu.sync_copy(zeros_hbm, zrows_v)
    pltpu.sync_copy(ones_hbm, ones_v)
    for r in range(ZC):
        pltpu.sync_copy(zrows_v, acc_sh.at[zid_v.at[r]])
    plsc.subcore_barrier()

    def body(j, carry):
        pltpu.sync_copy(ones_v, acc_sh.at[dst_v.at[j]], add=True)
        return carry

    lax.fori_loop(0, CHUNKS, body, 0)
    plsc.subcore_barrier()
    for r in range(ZC):
        pltpu.sync_copy(acc_sh.at[zid_v.at[r]], zrows_v)
        pltpu.sync_copy(zrows_v, out_hbm.at[pl.ds(c * NPAD + s * ZR + r * K, K)])


def _agg_body(h_hbm, src_hbm, dst_hbm, zid_hbm, zeros_hbm, out_hbm,
              src_v, dst_v, zid_v, rows_v, acc_sh, sem):
    c = lax.axis_index("c")
    s = lax.axis_index("s")
    wid = c * NS + s
    pltpu.sync_copy(src_hbm.at[wid], src_v)
    pltpu.sync_copy(dst_hbm.at[wid], dst_v)
    pltpu.sync_copy(zid_hbm.at[s], zid_v)
    pltpu.sync_copy(zeros_hbm, rows_v)
    for r in range(ZC):
        pltpu.sync_copy(rows_v, acc_sh.at[zid_v.at[r]])
    plsc.subcore_barrier()

    def body(j, carry):
        pltpu.async_copy(h_hbm.at[src_v.at[j]], rows_v, sem).wait()
        pltpu.sync_copy(rows_v, acc_sh.at[dst_v.at[j]], add=True)
        return carry

    lax.fori_loop(0, CHUNKS, body, 0)
    plsc.subcore_barrier()
    for r in range(ZC):
        pltpu.sync_copy(acc_sh.at[zid_v.at[r]], rows_v)
        pltpu.sync_copy(rows_v, out_hbm.at[pl.ds(c * NPAD + s * ZR + r * K, K)])


@functools.lru_cache(maxsize=1)
def _sc_kernels():
    mesh = plsc.VectorSubcoreMesh(core_axis_name="c", subcore_axis_name="s",
                                  num_cores=NC, num_subcores=NS)
    deg = pl.kernel(
        _deg_body,
        out_type=jax.ShapeDtypeStruct((NC * NPAD, D), jnp.float32),
        mesh=mesh,
        scratch_types=[
            pltpu.VMEM((CHUNKS, K), jnp.int32),
            pltpu.VMEM((ZC, K), jnp.int32),
            pltpu.VMEM((K, D), jnp.float32),
            pltpu.VMEM((K, D), jnp.float32),
            pltpu.VMEM_SHARED((NPAD, D), jnp.float32),
        ],
    )
    agg = pl.kernel(
        _agg_body,
        out_type=jax.ShapeDtypeStruct((NC * NPAD, D), jnp.float32),
        mesh=mesh,
        scratch_types=[
            pltpu.VMEM((CHUNKS, K), jnp.int32),
            pltpu.VMEM((CHUNKS, K), jnp.int32),
            pltpu.VMEM((ZC, K), jnp.int32),
            pltpu.VMEM((K, D), jnp.float32),
            pltpu.VMEM_SHARED((NPAD, D), jnp.float32),
            pltpu.SemaphoreType.DMA,
        ],
    )
    return deg, agg


def _tc1_body(deg0, deg1, x, W1, h1p, dinvc):
    deg = deg0[...][:, :1] + deg1[...][:, :1] + 1.0
    dinv = lax.rsqrt(deg)
    h = jnp.dot(x[...], W1[...], preferred_element_type=jnp.float32)
    h1p[...] = h * dinv
    dinvc[...] = dinv


def _tc2_body(a0, a1, hp, dinvc, b, W, out):
    dinv = dinvc[...]
    z = dinv * (a0[...] + a1[...] + hp[...]) + b[...]
    z = jnp.where(z >= 0, z, 0.01 * z)
    out[...] = jnp.dot(z, W[...], preferred_element_type=jnp.float32) * dinv


def _tc3_body(a0, a1, hp, dinvc, b, batch_r, Wfc, bfc, out, sums, cnt):
    i = pl.program_id(0)

    @pl.when(i == 0)
    def _():
        sums[...] = jnp.zeros_like(sums)
        cnt[...] = jnp.zeros_like(cnt)

    dinv = dinvc[...]
    z = dinv * (a0[...] + a1[...] + hp[...]) + b[...]
    z = jnp.where(z >= 0, z, 0.01 * z)
    seg = lax.broadcasted_iota(jnp.int32, (G, BLK), 0)
    m = (batch_r[...].reshape(1, BLK) == seg).astype(jnp.float32)
    sums[...] += jnp.dot(m, z, preferred_element_type=jnp.float32)
    cnt[...] += jnp.broadcast_to(jnp.sum(m, axis=1, keepdims=True), (G, D))

    @pl.when(i == pl.num_programs(0) - 1)
    def _():
        pooled = sums[...] / jnp.maximum(cnt[...], 1.0)
        out[...] = jnp.dot(pooled, Wfc[...], preferred_element_type=jnp.float32) + bfc[...]


def _row_spec(width):
    return pl.BlockSpec((BLK, width), lambda i: (i, 0))


def _full_spec(shape):
    return pl.BlockSpec(shape, lambda i: tuple(0 for _ in shape))


_tc1 = pl.pallas_call(
    _tc1_body,
    grid=(N // BLK,),
    in_specs=[_row_spec(D), _row_spec(D), _row_spec(D), _full_spec((D, D))],
    out_specs=[_row_spec(D), _row_spec(1)],
    out_shape=[
        jax.ShapeDtypeStruct((N, D), jnp.float32),
        jax.ShapeDtypeStruct((N, 1), jnp.float32),
    ],
)

_tc2 = pl.pallas_call(
    _tc2_body,
    grid=(N // BLK,),
    in_specs=[_row_spec(D), _row_spec(D), _row_spec(D), _row_spec(1),
              _full_spec((1, D)), _full_spec((D, D))],
    out_specs=_row_spec(D),
    out_shape=jax.ShapeDtypeStruct((N, D), jnp.float32),
)

_tc3 = pl.pallas_call(
    _tc3_body,
    grid=(N // BLK,),
    in_specs=[_row_spec(D), _row_spec(D), _row_spec(D), _row_spec(1),
              _full_spec((1, D)), pl.BlockSpec((1, 1, BLK), lambda i: (i, 0, 0)),
              _full_spec((D, 1)), _full_spec((1, 1))],
    out_specs=_full_spec((G, 1)),
    out_shape=jax.ShapeDtypeStruct((G, 1), jnp.float32),
    scratch_shapes=[
        pltpu.VMEM((G, D), jnp.float32),
        pltpu.VMEM((G, D), jnp.float32),
    ],
)


def kernel(x, edge_index, batch, W1, b1, W2, b2, Wfc, bfc):
    src = edge_index[0]
    dst = edge_index[1]
    npad_e = EP - E
    pad_src = jnp.zeros((npad_e,), jnp.int32)
    pad_dst = N + (jnp.arange(npad_e, dtype=jnp.int32) % (NPAD - N))
    srcp = jnp.concatenate([src, pad_src]).reshape(NW, CHUNKS, K)
    dstp = jnp.concatenate([dst, pad_dst]).reshape(NW, CHUNKS, K)

    zid = (jnp.arange(NS * ZR, dtype=jnp.int32)).reshape(NS, ZC, K)
    zerosD = jnp.zeros((K, D), jnp.float32)
    onesD = jnp.ones((K, D), jnp.float32)

    deg_kernel, agg_kernel = _sc_kernels()
    degs = deg_kernel(dstp, zid, zerosD, onesD)
    h1p, dinvc = _tc1(degs[:NPAD], degs[NPAD:], x, W1)

    acc1 = agg_kernel(h1p, srcp, dstp, zid, zerosD)
    h2p = _tc2(acc1[:NPAD], acc1[NPAD:], h1p, dinvc, b1.reshape(1, D), W2)

    acc2 = agg_kernel(h2p, srcp, dstp, zid, zerosD)
    return _tc3(acc2[:NPAD], acc2[NPAD:], h2p, dinvc, b2.reshape(1, D),
                batch.reshape(N // BLK, 1, BLK), Wfc, bfc.reshape(1, 1))


# speedup vs baseline: 8.7804x; 1.1079x over previous
import functools

import jax
import jax.numpy as jnp
from jax import lax
from jax.experimental import pallas as pl
from jax.experimental.pallas import tpu as pltpu
from jax.experimental.pallas import tpu_sc as plsc

N = 10000
E = 320000
D = 128
G = 16

NC = 2
NS = 16
NW = NC * NS

K = 128
CHUNKS = 80
EP = NW * CHUNKS * K
NPAD = 10240
ZR = NPAD // NS
ZC = ZR // K

BLK = 1000


def _deg_body(dst_hbm, zid_hbm, zeros_hbm, ones_hbm, out_hbm,
              dst_v, zid_v, zrows_v, ones_v, acc_sh):
    c = lax.axis_index("c")
    s = lax.axis_index("s")
    wid = c * NS + s
    pltpu.sync_copy(dst_hbm.at[wid], dst_v)
    pltpu.sync_copy(zid_hbm.at[s], zid_v)
    pltpu.sync_copy(zeros_hbm, zrows_v)
    pltpu.sync_copy(ones_hbm, ones_v)
    for r in range(ZC):
        pltpu.sync_copy(zrows_v, acc_sh.at[zid_v.at[r]])
    plsc.subcore_barrier()

    def body(j, carry):
        pltpu.sync_copy(ones_v, acc_sh.at[dst_v.at[j]], add=True)
        return carry

    lax.fori_loop(0, CHUNKS, body, 0)
    plsc.subcore_barrier()
    for r in range(ZC):
        pltpu.sync_copy(acc_sh.at[zid_v.at[r]], zrows_v)
        pltpu.sync_copy(zrows_v, out_hbm.at[pl.ds(c * NPAD + s * ZR + r * K, K)])


def _agg_body(h_hbm, src_hbm, dst_hbm, zid_hbm, zeros_hbm, out_hbm,
              src_v, dsti_v, zid_v, rows_a, rows_b,
              acc_sh, sem_ra, sem_rb, sem_ia, sem_ib):
    c = lax.axis_index("c")
    s = lax.axis_index("s")
    wid = c * NS + s
    pltpu.sync_copy(src_hbm.at[wid], src_v)
    pltpu.sync_copy(zid_hbm.at[s], zid_v)
    pltpu.sync_copy(zeros_hbm, rows_a)
    for r in range(ZC):
        pltpu.sync_copy(rows_a, acc_sh.at[zid_v.at[r]])
    plsc.subcore_barrier()

    pltpu.async_copy(h_hbm.at[src_v.at[0]], rows_a, sem_ra)
    pltpu.async_copy(dst_hbm.at[wid, 0], dsti_v.at[0], sem_ia)

    def body(t, carry):
        j0 = 2 * t
        pltpu.async_copy(h_hbm.at[src_v.at[j0 + 1]], rows_b, sem_rb)
        pltpu.async_copy(dst_hbm.at[wid, j0 + 1], dsti_v.at[1], sem_ib)
        pltpu.make_async_copy(h_hbm.at[src_v.at[0]], rows_a, sem_ra).wait()
        pltpu.make_async_copy(dst_hbm.at[wid, 0], dsti_v.at[0], sem_ia).wait()
        pltpu.sync_copy(rows_a, acc_sh.at[dsti_v.at[0]], add=True)
        j2 = (j0 + 2) % CHUNKS
        pltpu.async_copy(h_hbm.at[src_v.at[j2]], rows_a, sem_ra)
        pltpu.async_copy(dst_hbm.at[wid, j2], dsti_v.at[0], sem_ia)
        pltpu.make_async_copy(h_hbm.at[src_v.at[0]], rows_b, sem_rb).wait()
        pltpu.make_async_copy(dst_hbm.at[wid, 0], dsti_v.at[1], sem_ib).wait()
        pltpu.sync_copy(rows_b, acc_sh.at[dsti_v.at[1]], add=True)
        return carry

    lax.fori_loop(0, CHUNKS // 2, body, 0)
    pltpu.make_async_copy(h_hbm.at[src_v.at[0]], rows_a, sem_ra).wait()
    pltpu.make_async_copy(dst_hbm.at[wid, 0], dsti_v.at[0], sem_ia).wait()
    plsc.subcore_barrier()
    for r in range(ZC):
        pltpu.sync_copy(acc_sh.at[zid_v.at[r]], rows_a)
        pltpu.sync_copy(rows_a, out_hbm.at[pl.ds(c * NPAD + s * ZR + r * K, K)])


@functools.lru_cache(maxsize=1)
def _sc_kernels():
    mesh = plsc.VectorSubcoreMesh(core_axis_name="c", subcore_axis_name="s",
                                  num_cores=NC, num_subcores=NS)
    deg = pl.kernel(
        _deg_body,
        out_type=jax.ShapeDtypeStruct((NC * NPAD, D), jnp.float32),
        mesh=mesh,
        scratch_types=[
            pltpu.VMEM((CHUNKS, K), jnp.int32),
            pltpu.VMEM((ZC, K), jnp.int32),
            pltpu.VMEM((K, D), jnp.float32),
            pltpu.VMEM((K, D), jnp.float32),
            pltpu.VMEM_SHARED((NPAD, D), jnp.float32),
        ],
    )
    agg = pl.kernel(
        _agg_body,
        out_type=jax.ShapeDtypeStruct((NC * NPAD, D), jnp.float32),
        mesh=mesh,
        scratch_types=[
            pltpu.VMEM((CHUNKS, K), jnp.int32),
            pltpu.VMEM((2, K), jnp.int32),
            pltpu.VMEM((ZC, K), jnp.int32),
            pltpu.VMEM((K, D), jnp.float32),
            pltpu.VMEM((K, D), jnp.float32),
            pltpu.VMEM_SHARED((NPAD, D), jnp.float32),
            pltpu.SemaphoreType.DMA,
            pltpu.SemaphoreType.DMA,
            pltpu.SemaphoreType.DMA,
            pltpu.SemaphoreType.DMA,
        ],
    )
    return deg, agg


def _tc1_body(deg0, deg1, x, W1, h1p, dinvc):
    deg = deg0[...][:, :1] + deg1[...][:, :1] + 1.0
    dinv = lax.rsqrt(deg)
    h = jnp.dot(x[...], W1[...], preferred_element_type=jnp.float32)
    h1p[...] = h * dinv
    dinvc[...] = dinv


def _tc2_body(a0, a1, hp, dinvc, b, W, out):
    dinv = dinvc[...]
    z = dinv * (a0[...] + a1[...] + hp[...]) + b[...]
    z = jnp.where(z >= 0, z, 0.01 * z)
    out[...] = jnp.dot(z, W[...], preferred_element_type=jnp.float32) * dinv


def _tc3_body(a0, a1, hp, dinvc, b, batch_r, Wfc, bfc, out, sums, cnt):
    i = pl.program_id(0)

    @pl.when(i == 0)
    def _():
        sums[...] = jnp.zeros_like(sums)
        cnt[...] = jnp.zeros_like(cnt)

    dinv = dinvc[...]
    z = dinv * (a0[...] + a1[...] + hp[...]) + b[...]
    z = jnp.where(z >= 0, z, 0.01 * z)
    seg = lax.broadcasted_iota(jnp.int32, (G, BLK), 0)
    m = (batch_r[...].reshape(1, BLK) == seg).astype(jnp.float32)
    sums[...] += jnp.dot(m, z, preferred_element_type=jnp.float32)
    cnt[...] += jnp.broadcast_to(jnp.sum(m, axis=1, keepdims=True), (G, D))

    @pl.when(i == pl.num_programs(0) - 1)
    def _():
        pooled = sums[...] / jnp.maximum(cnt[...], 1.0)
        out[...] = jnp.dot(pooled, Wfc[...], preferred_element_type=jnp.float32) + bfc[...]


def _row_spec(width):
    return pl.BlockSpec((BLK, width), lambda i: (i, 0))


def _full_spec(shape):
    return pl.BlockSpec(shape, lambda i: tuple(0 for _ in shape))


_tc1 = pl.pallas_call(
    _tc1_body,
    grid=(N // BLK,),
    in_specs=[_row_spec(D), _row_spec(D), _row_spec(D), _full_spec((D, D))],
    out_specs=[_row_spec(D), _row_spec(1)],
    out_shape=[
        jax.ShapeDtypeStruct((N, D), jnp.float32),
        jax.ShapeDtypeStruct((N, 1), jnp.float32),
    ],
)

_tc2 = pl.pallas_call(
    _tc2_body,
    grid=(N // BLK,),
    in_specs=[_row_spec(D), _row_spec(D), _row_spec(D), _row_spec(1),
              _full_spec((1, D)), _full_spec((D, D))],
    out_specs=_row_spec(D),
    out_shape=jax.ShapeDtypeStruct((N, D), jnp.float32),
)

_tc3 = pl.pallas_call(
    _tc3_body,
    grid=(N // BLK,),
    in_specs=[_row_spec(D), _row_spec(D), _row_spec(D), _row_spec(1),
              _full_spec((1, D)), pl.BlockSpec((1, 1, BLK), lambda i: (i, 0, 0)),
              _full_spec((D, 1)), _full_spec((1, 1))],
    out_specs=_full_spec((G, 1)),
    out_shape=jax.ShapeDtypeStruct((G, 1), jnp.float32),
    scratch_shapes=[
        pltpu.VMEM((G, D), jnp.float32),
        pltpu.VMEM((G, D), jnp.float32),
    ],
)


def kernel(x, edge_index, batch, W1, b1, W2, b2, Wfc, bfc):
    src = edge_index[0]
    dst = edge_index[1]
    npad_e = EP - E
    pad_src = jnp.zeros((npad_e,), jnp.int32)
    pad_dst = N + (jnp.arange(npad_e, dtype=jnp.int32) % (NPAD - N))
    srcp = jnp.concatenate([src, pad_src]).reshape(NW, CHUNKS, K)
    dstp = jnp.concatenate([dst, pad_dst]).reshape(NW, CHUNKS, K)

    zid = (jnp.arange(NS * ZR, dtype=jnp.int32)).reshape(NS, ZC, K)
    zerosD = jnp.zeros((K, D), jnp.float32)
    onesD = jnp.ones((K, D), jnp.float32)

    deg_kernel, agg_kernel = _sc_kernels()
    degs = deg_kernel(dstp, zid, zerosD, onesD)
    h1p, dinvc = _tc1(degs[:NPAD], degs[NPAD:], x, W1)

    acc1 = agg_kernel(h1p, srcp, dstp, zid, zerosD)
    h2p = _tc2(acc1[:NPAD], acc1[NPAD:], h1p, dinvc, b1.reshape(1, D), W2)

    acc2 = agg_kernel(h2p, srcp, dstp, zid, zerosD)
    return _tc3(acc2[:NPAD], acc2[NPAD:], h2p, dinvc, b2.reshape(1, D),
                batch.reshape(N // BLK, 1, BLK), Wfc, bfc.reshape(1, 1))


# speedup vs baseline: 9.1441x; 1.0414x over previous
import functools

import jax
import jax.numpy as jnp
from jax import lax
from jax.experimental import pallas as pl
from jax.experimental.pallas import tpu as pltpu
from jax.experimental.pallas import tpu_sc as plsc

N = 10000
E = 320000
D = 128
G = 16

NC = 2
NS = 16
NW = NC * NS

K = 128
CHUNKS = 80
EP = NW * CHUNKS * K
NPAD = 10240
ZR = NPAD // NS
ZC = ZR // K

BLK = 1000


def _deg_body(dst_hbm, zid_hbm, zeros_hbm, ones_hbm, out_hbm,
              dst_v, zid_v, zrows_v, ones_v, acc_sh):
    c = lax.axis_index("c")
    s = lax.axis_index("s")
    wid = c * NS + s
    pltpu.sync_copy(dst_hbm.at[wid], dst_v)
    pltpu.sync_copy(zid_hbm.at[s], zid_v)
    pltpu.sync_copy(zeros_hbm, zrows_v)
    pltpu.sync_copy(ones_hbm, ones_v)
    for r in range(ZC):
        pltpu.sync_copy(zrows_v, acc_sh.at[zid_v.at[r]])
    plsc.subcore_barrier()

    def body(j, carry):
        pltpu.sync_copy(ones_v, acc_sh.at[dst_v.at[j]], add=True)
        return carry

    lax.fori_loop(0, CHUNKS, body, 0)
    plsc.subcore_barrier()
    for r in range(ZC):
        pltpu.sync_copy(acc_sh.at[zid_v.at[r]], zrows_v)
        pltpu.sync_copy(zrows_v, out_hbm.at[pl.ds(c * NPAD + s * ZR + r * K, K)])


def _agg_body(h_hbm, src_hbm, dst_hbm, zid_hbm, zeros_hbm, out_hbm,
              src_v, dsti_v, zid_v, rows_a, rows_b,
              acc_sh, sem_ra, sem_rb, sem_ia, sem_ib):
    c = lax.axis_index("c")
    s = lax.axis_index("s")
    wid = c * NS + s
    pltpu.sync_copy(src_hbm.at[wid], src_v)
    pltpu.sync_copy(zid_hbm.at[s], zid_v)
    pltpu.sync_copy(zeros_hbm, rows_a)
    for r in range(ZC):
        pltpu.sync_copy(rows_a, acc_sh.at[zid_v.at[r]])
    plsc.subcore_barrier()

    pltpu.async_copy(h_hbm.at[src_v.at[0]], rows_a, sem_ra)
    pltpu.async_copy(dst_hbm.at[wid, 0], dsti_v.at[0], sem_ia)

    def body(t, carry):
        j0 = 2 * t
        pltpu.async_copy(h_hbm.at[src_v.at[j0 + 1]], rows_b, sem_rb)
        pltpu.async_copy(dst_hbm.at[wid, j0 + 1], dsti_v.at[1], sem_ib)
        pltpu.make_async_copy(h_hbm.at[src_v.at[0]], rows_a, sem_ra).wait()
        pltpu.make_async_copy(dst_hbm.at[wid, 0], dsti_v.at[0], sem_ia).wait()
        pltpu.sync_copy(rows_a, acc_sh.at[dsti_v.at[0]], add=True)
        j2 = (j0 + 2) % CHUNKS
        pltpu.async_copy(h_hbm.at[src_v.at[j2]], rows_a, sem_ra)
        pltpu.async_copy(dst_hbm.at[wid, j2], dsti_v.at[0], sem_ia)
        pltpu.make_async_copy(h_hbm.at[src_v.at[0]], rows_b, sem_rb).wait()
        pltpu.make_async_copy(dst_hbm.at[wid, 0], dsti_v.at[1], sem_ib).wait()
        pltpu.sync_copy(rows_b, acc_sh.at[dsti_v.at[1]], add=True)
        return carry

    lax.fori_loop(0, CHUNKS // 2, body, 0)
    pltpu.make_async_copy(h_hbm.at[src_v.at[0]], rows_a, sem_ra).wait()
    pltpu.make_async_copy(dst_hbm.at[wid, 0], dsti_v.at[0], sem_ia).wait()
    plsc.subcore_barrier()
    for r in range(ZC):
        pltpu.sync_copy(acc_sh.at[zid_v.at[r]], rows_a)
        pltpu.sync_copy(rows_a, out_hbm.at[pl.ds(c * NPAD + s * ZR + r * K, K)])


@functools.lru_cache(maxsize=1)
def _sc_kernels():
    mesh = plsc.VectorSubcoreMesh(core_axis_name="c", subcore_axis_name="s",
                                  num_cores=NC, num_subcores=NS)
    deg = pl.kernel(
        _deg_body,
        out_type=jax.ShapeDtypeStruct((NC * NPAD, D), jnp.float32),
        mesh=mesh,
        scratch_types=[
            pltpu.VMEM((CHUNKS, K), jnp.int32),
            pltpu.VMEM((ZC, K), jnp.int32),
            pltpu.VMEM((K, D), jnp.float32),
            pltpu.VMEM((K, D), jnp.float32),
            pltpu.VMEM_SHARED((NPAD, D), jnp.float32),
        ],
    )
    agg = pl.kernel(
        _agg_body,
        out_type=jax.ShapeDtypeStruct((NC * NPAD, D), jnp.float32),
        mesh=mesh,
        scratch_types=[
            pltpu.VMEM((CHUNKS, K), jnp.int32),
            pltpu.VMEM((2, K), jnp.int32),
            pltpu.VMEM((ZC, K), jnp.int32),
            pltpu.VMEM((K, D), jnp.float32),
            pltpu.VMEM((K, D), jnp.float32),
            pltpu.VMEM_SHARED((NPAD, D), jnp.float32),
            pltpu.SemaphoreType.DMA,
            pltpu.SemaphoreType.DMA,
            pltpu.SemaphoreType.DMA,
            pltpu.SemaphoreType.DMA,
        ],
    )
    return deg, agg


def _tc0_body(x, W1, h1):
    h1[...] = jnp.dot(x[...], W1[...], preferred_element_type=jnp.float32)


def _tc1_body(deg0, deg1, h, h1p, dinvc):
    deg = deg0[...][:, :1] + deg1[...][:, :1] + 1.0
    dinv = lax.rsqrt(deg)
    h1p[...] = h[...] * dinv
    dinvc[...] = dinv


def _tc2_body(a0, a1, hp, dinvc, b, W, out):
    dinv = dinvc[...]
    z = dinv * (a0[...] + a1[...] + hp[...]) + b[...]
    z = jnp.where(z >= 0, z, 0.01 * z)
    out[...] = jnp.dot(z, W[...], preferred_element_type=jnp.float32) * dinv


def _tc3_body(a0, a1, hp, dinvc, b, batch_r, Wfc, bfc, out, sums, cnt):
    i = pl.program_id(0)

    @pl.when(i == 0)
    def _():
        sums[...] = jnp.zeros_like(sums)
        cnt[...] = jnp.zeros_like(cnt)

    dinv = dinvc[...]
    z = dinv * (a0[...] + a1[...] + hp[...]) + b[...]
    z = jnp.where(z >= 0, z, 0.01 * z)
    seg = lax.broadcasted_iota(jnp.int32, (G, BLK), 0)
    m = (batch_r[...].reshape(1, BLK) == seg).astype(jnp.float32)
    sums[...] += jnp.dot(m, z, preferred_element_type=jnp.float32)
    cnt[...] += jnp.broadcast_to(jnp.sum(m, axis=1, keepdims=True), (G, D))

    @pl.when(i == pl.num_programs(0) - 1)
    def _():
        pooled = sums[...] / jnp.maximum(cnt[...], 1.0)
        out[...] = jnp.dot(pooled, Wfc[...], preferred_element_type=jnp.float32) + bfc[...]


def _row_spec(width):
    return pl.BlockSpec((BLK, width), lambda i: (i, 0))


def _full_spec(shape):
    return pl.BlockSpec(shape, lambda i: tuple(0 for _ in shape))


_tc0 = pl.pallas_call(
    _tc0_body,
    grid=(N // BLK,),
    in_specs=[_row_spec(D), _full_spec((D, D))],
    out_specs=_row_spec(D),
    out_shape=jax.ShapeDtypeStruct((N, D), jnp.float32),
)

_tc1 = pl.pallas_call(
    _tc1_body,
    grid=(N // BLK,),
    in_specs=[_row_spec(D), _row_spec(D), _row_spec(D)],
    out_specs=[_row_spec(D), _row_spec(1)],
    out_shape=[
        jax.ShapeDtypeStruct((N, D), jnp.float32),
        jax.ShapeDtypeStruct((N, 1), jnp.float32),
    ],
)

_tc2 = pl.pallas_call(
    _tc2_body,
    grid=(N // BLK,),
    in_specs=[_row_spec(D), _row_spec(D), _row_spec(D), _row_spec(1),
              _full_spec((1, D)), _full_spec((D, D))],
    out_specs=_row_spec(D),
    out_shape=jax.ShapeDtypeStruct((N, D), jnp.float32),
)

_tc3 = pl.pallas_call(
    _tc3_body,
    grid=(N // BLK,),
    in_specs=[_row_spec(D), _row_spec(D), _row_spec(D), _row_spec(1),
              _full_spec((1, D)), pl.BlockSpec((1, 1, BLK), lambda i: (i, 0, 0)),
              _full_spec((D, 1)), _full_spec((1, 1))],
    out_specs=_full_spec((G, 1)),
    out_shape=jax.ShapeDtypeStruct((G, 1), jnp.float32),
    scratch_shapes=[
        pltpu.VMEM((G, D), jnp.float32),
        pltpu.VMEM((G, D), jnp.float32),
    ],
)


def kernel(x, edge_index, batch, W1, b1, W2, b2, Wfc, bfc):
    src = edge_index[0]
    dst = edge_index[1]
    npad_e = EP - E
    pad_src = jnp.zeros((npad_e,), jnp.int32)
    pad_dst = N + (jnp.arange(npad_e, dtype=jnp.int32) % (NPAD - N))
    srcp = jnp.concatenate([src, pad_src]).reshape(NW, CHUNKS, K)
    dstp = jnp.concatenate([dst, pad_dst]).reshape(NW, CHUNKS, K)

    zid = (jnp.arange(NS * ZR, dtype=jnp.int32)).reshape(NS, ZC, K)
    zerosD = jnp.zeros((K, D), jnp.float32)
    onesD = jnp.ones((K, D), jnp.float32)

    deg_kernel, agg_kernel = _sc_kernels()
    h1 = _tc0(x, W1)
    degs = deg_kernel(dstp, zid, zerosD, onesD)
    h1p, dinvc = _tc1(degs[:NPAD], degs[NPAD:], h1)

    acc1 = agg_kernel(h1p, srcp, dstp, zid, zerosD)
    h2p = _tc2(acc1[:NPAD], acc1[NPAD:], h1p, dinvc, b1.reshape(1, D), W2)

    acc2 = agg_kernel(h2p, srcp, dstp, zid, zerosD)
    return _tc3(acc2[:NPAD], acc2[NPAD:], h2p, dinvc, b2.reshape(1, D),
                batch.reshape(N // BLK, 1, BLK), Wfc, bfc.reshape(1, 1))
